# baseline (speedup 1.0000x reference)
import functools
import jax
import jax.numpy as jnp
from jax import lax
from jax.experimental import pallas as pl
from jax.experimental.pallas import tpu as pltpu, tpu_sc as plsc

CH = 64


def kernel(inputs, table):
    b, l = inputs.shape
    d = table.shape[1]
    nw = 32
    rows_per_w = l // nw
    mesh = plsc.VectorSubcoreMesh(core_axis_name="c", subcore_axis_name="s")

    @functools.partial(
        pl.kernel,
        mesh=mesh,
        out_type=jax.ShapeDtypeStruct((b, l, d), table.dtype),
        scratch_types=[
            pltpu.VMEM((CH, d), table.dtype),
            pltpu.SemaphoreType.DMA,
        ],
    )
    def k(table_hbm, out_hbm, buf, sem):
        wid = lax.axis_index("s") * 2 + lax.axis_index("c")
        base = wid * rows_per_w
        for c in range(rows_per_w // CH):
            start = base + c * CH
            pltpu.sync_copy(table_hbm.at[pl.ds(start, CH)], buf)
            copies = [
                pltpu.async_copy(buf, out_hbm.at[bi, pl.ds(start, CH)], sem)
                for bi in range(b)
            ]
            for cp in copies:
                cp.wait()

    return k(table)


# speedup vs baseline: 1.8312x; 1.8312x over previous
import jax
import jax.numpy as jnp
from jax.experimental import pallas as pl
from jax.experimental.pallas import tpu as pltpu

NCH = 4


def _body(table_ref, out_ref, *scratch):
    bufs = scratch[:NCH]
    sem_in = scratch[NCH]
    sem_out = scratch[NCH + 1]
    b, l, d = out_ref.shape
    ch = l // NCH
    in_cps = []
    for c in range(NCH):
        cp = pltpu.make_async_copy(
            table_ref.at[pl.ds(c * ch, ch)], bufs[c], sem_in.at[c]
        )
        cp.start()
        in_cps.append(cp)
    out_cps = []
    for c in range(NCH):
        in_cps[c].wait()
        for bi in range(b):
            cp = pltpu.make_async_copy(
                bufs[c], out_ref.at[bi, pl.ds(c * ch, ch)], sem_out
            )
            cp.start()
            out_cps.append(cp)
    for cp in out_cps:
        cp.wait()


def kernel(inputs, table):
    b, l = inputs.shape
    d = table.shape[1]
    ch = l // NCH
    return pl.pallas_call(
        _body,
        in_specs=[pl.BlockSpec(memory_space=pltpu.MemorySpace.HBM)],
        out_specs=pl.BlockSpec(memory_space=pltpu.MemorySpace.HBM),
        out_shape=jax.ShapeDtypeStruct((b, l, d), table.dtype),
        scratch_shapes=(
            [pltpu.VMEM((ch, d), table.dtype) for _ in range(NCH)]
            + [pltpu.SemaphoreType.DMA((NCH,)), pltpu.SemaphoreType.DMA]
        ),
    )(table)
